# baseline (speedup 1.0000x reference)
import functools

import jax
import jax.numpy as jnp
from jax import lax
from jax.experimental import pallas as pl
from jax.experimental.pallas import tpu as pltpu
from jax.experimental.pallas import tpu_sc as plsc

B = 16384
DMF = 16
DMLP = 32

NC = 2
NS = 16
NW = NC * NS
CHUNK = 128
CH = B // (NW * CHUNK)

@functools.lru_cache(maxsize=None)
def _build_sc_gather():
    mesh = plsc.VectorSubcoreMesh(core_axis_name="c", subcore_axis_name="s")

    @functools.partial(
        pl.kernel,
        out_type=(
            jax.ShapeDtypeStruct((NW, CH, CHUNK, DMLP), jnp.float32),
            jax.ShapeDtypeStruct((NW, CH, CHUNK, DMLP), jnp.float32),
            jax.ShapeDtypeStruct((NW, CH, CHUNK, DMF), jnp.float32),
            jax.ShapeDtypeStruct((NW, CH, CHUNK, DMF), jnp.float32),
        ),
        mesh=mesh,
        compiler_params=pltpu.CompilerParams(use_tc_tiling_on_sc=False),
        scratch_types=(
            pltpu.VMEM((CH, CHUNK), jnp.int32),
            pltpu.VMEM((CH, CHUNK), jnp.int32),
            pltpu.VMEM((CH, CHUNK, DMLP), jnp.float32),
            pltpu.VMEM((CH, CHUNK, DMLP), jnp.float32),
            pltpu.VMEM((CH, CHUNK, DMF), jnp.float32),
            pltpu.VMEM((CH, CHUNK, DMF), jnp.float32),
            pltpu.SemaphoreType.DMA,
        ),
    )
    def _sc_gather(uidx_hbm, iidx_hbm, t_umlp, t_imlp, t_umf, t_imf,
                   o_umlp, o_imlp, o_umf, o_imf,
                   uidx_v, iidx_v, b_umlp, b_imlp, b_umf, b_imf, sem):
        wid = lax.axis_index("s") * NC + lax.axis_index("c")
        pltpu.sync_copy(uidx_hbm.at[wid], uidx_v)
        pltpu.sync_copy(iidx_hbm.at[wid], iidx_v)
        cps = []
        for j in range(CH):
            cps.append(pltpu.async_copy(t_umlp.at[uidx_v.at[j]], b_umlp.at[j], sem))
            cps.append(pltpu.async_copy(t_imlp.at[iidx_v.at[j]], b_imlp.at[j], sem))
            cps.append(pltpu.async_copy(t_umf.at[uidx_v.at[j]], b_umf.at[j], sem))
            cps.append(pltpu.async_copy(t_imf.at[iidx_v.at[j]], b_imf.at[j], sem))
        for cp in cps:
            cp.wait()
        pltpu.sync_copy(b_umlp, o_umlp.at[wid])
        pltpu.sync_copy(b_imlp, o_imlp.at[wid])
        pltpu.sync_copy(b_umf, o_umf.at[wid])
        pltpu.sync_copy(b_imf, o_imf.at[wid])

    return _sc_gather


BT = 2048


def _leaky(x):
    return jnp.where(x >= 0, x, 0.01 * x)


def _tc_head(xu_ref, xi_ref, mu_ref, mi_ref, w0u_ref, w0i_ref, b0_ref,
             w1_ref, b1_ref, cwm_ref, cwf_ref, cb_ref, aw_ref, ab_ref,
             out_ref):
    hi = lax.Precision.HIGHEST
    h0 = (jnp.dot(xu_ref[...], w0u_ref[...], precision=hi)
          + jnp.dot(xi_ref[...], w0i_ref[...], precision=hi)
          + b0_ref[...])
    h0 = _leaky(h0)
    h1 = _leaky(jnp.dot(h0, w1_ref[...], precision=hi) + b1_ref[...])
    mf = mu_ref[...] * mi_ref[...]
    v = _leaky(jnp.dot(h1, cwm_ref[...], precision=hi)
               + jnp.dot(mf, cwf_ref[...], precision=hi)
               + cb_ref[...])
    logit = jnp.sum(v * aw_ref[...], axis=1, keepdims=True) + ab_ref[...]
    out_ref[...] = 1.0 / (1.0 + jnp.exp(-logit))


def _full(shape):
    return pl.BlockSpec(shape, lambda i: (0, 0))


_mlp_head = pl.pallas_call(
    _tc_head,
    grid=(B // BT,),
    in_specs=[
        pl.BlockSpec((BT, DMLP), lambda i: (i, 0)),
        pl.BlockSpec((BT, DMLP), lambda i: (i, 0)),
        pl.BlockSpec((BT, DMF), lambda i: (i, 0)),
        pl.BlockSpec((BT, DMF), lambda i: (i, 0)),
        _full((DMLP, 128)),
        _full((DMLP, 128)),
        _full((1, 128)),
        _full((128, 64)),
        _full((1, 64)),
        _full((64, 64)),
        _full((DMF, 64)),
        _full((1, 64)),
        _full((1, 64)),
        _full((1, 1)),
    ],
    out_specs=pl.BlockSpec((BT, 1), lambda i: (i, 0)),
    out_shape=jax.ShapeDtypeStruct((B, 1), jnp.float32),
)


def kernel(user_indices, item_indices, emb_acc_mlp, emb_loc_mlp,
           emb_acc_mf, emb_loc_mf, fc0_w, fc0_b, fc1_w, fc1_b,
           common_w, common_b, aff_w, aff_b):
    uidx = user_indices.astype(jnp.int32).reshape(NW, CH, CHUNK)
    iidx = item_indices.astype(jnp.int32).reshape(NW, CH, CHUNK)
    u_mlp, i_mlp, u_mf, i_mf = _build_sc_gather()(
        uidx, iidx, emb_acc_mlp, emb_loc_mlp, emb_acc_mf, emb_loc_mf)
    u_mlp = u_mlp.reshape(B, DMLP)
    i_mlp = i_mlp.reshape(B, DMLP)
    u_mf = u_mf.reshape(B, DMF)
    i_mf = i_mf.reshape(B, DMF)

    w0 = fc0_w.T
    w0u, w0i = w0[:DMLP], w0[DMLP:]
    w1 = fc1_w.T
    cw = common_w.T
    cwm, cwf = cw[:64], cw[64:]
    return _mlp_head(
        u_mlp, i_mlp, u_mf, i_mf,
        w0u, w0i, fc0_b.reshape(1, 128),
        w1, fc1_b.reshape(1, 64),
        cwm, cwf, common_b.reshape(1, 64),
        aff_w, aff_b.reshape(1, 1))
